# baseline (speedup 1.0000x reference)
import functools

import jax
import jax.numpy as jnp
from jax import lax
from jax.experimental import pallas as pl
from jax.experimental.pallas import tpu as pltpu
from jax.experimental.pallas import tpu_sc as plsc


def _tc_kernel(feats_ref, lang_ref, centers_ref, sizes_ref, centersF_ref,
               w1a_ref, w1b_ref, w1gp_ref, w1gs_ref, w1l_ref, b1_ref,
               base2_ref, g2_ref, featsp_ref, idx_ref, fidx_ref):
    T = centers_ref.shape[1]
    N = centersF_ref.shape[1]
    K = idx_ref.shape[2]
    Dpad = featsp_ref.shape[2]
    b = pl.program_id(0)
    r0 = pl.program_id(1) * T

    f = feats_ref[0]
    fa = jnp.dot(f, w1a_ref[...], preferred_element_type=jnp.float32)
    fb = jnp.dot(f, w1b_ref[...], preferred_element_type=jnp.float32)
    c = centers_ref[0]
    s = sizes_ref[0]
    cs = c[:, 0:1] * w1gp_ref[0:1, :] + s[:, 0:1] * w1gs_ref[0:1, :]
    for dd in range(1, 3):
        cs = cs + c[:, dd:dd + 1] * w1gp_ref[dd:dd + 1, :]
        cs = cs + s[:, dd:dd + 1] * w1gs_ref[dd:dd + 1, :]
    lb = jnp.dot(lang_ref[0], w1l_ref[...],
                 preferred_element_type=jnp.float32)
    base2_ref[0] = fa + cs + (lb + b1_ref[...])
    g2_ref[0] = fb - cs
    featsp_ref[0] = jnp.concatenate(
        [f, jnp.zeros((T, Dpad - f.shape[1]), jnp.float32)], axis=1)

    cf = centersF_ref[0]
    acc = None
    for dd in range(3):
        diff = c[:, dd:dd + 1] - cf[:, dd:dd + 1].T
        sq = diff * diff
        acc = sq if acc is None else acc + sq
    col = jax.lax.broadcasted_iota(jnp.int32, (T, N), 1)
    row_g = r0 + jax.lax.broadcasted_iota(jnp.int32, (T, N), 0)
    d = jnp.where(col == row_g, jnp.inf, acc)

    idxs = []
    for _ in range(K):
        m = jnp.min(d, axis=1, keepdims=True)
        cand = jnp.where(d == m, col, N)
        aj = jnp.min(cand, axis=1, keepdims=True)
        idxs.append(aj)
        d = jnp.where(col == aj, jnp.inf, d)
    idx = jnp.concatenate(idxs, axis=1)
    idx_ref[0] = idx
    fidx_ref[0] = idx + b * N


def _sc_body(g2_hbm, feats_hbm, featsp_hbm, base2_hbm, idx_hbm, w2_hbm,
             enh_hbm, wout_hbm,
             idx0_v, idx1_v, g2r0_v, g2r1_v, fr0_v, fr1_v,
             b_v, fs_v, out_v, wch_v, w2_v,
             semg0, semg1, semf0, semf1,
             *, rows_per_w, ch, h_dim, d_dim, k):
    nc = 2
    wid = lax.axis_index("s") * nc + lax.axis_index("c")
    base0 = wid * rows_per_w
    nch = rows_per_w // ch
    sets = ((idx0_v, g2r0_v, fr0_v, semg0, semf0),
            (idx1_v, g2r1_v, fr1_v, semg1, semf1))

    pltpu.sync_copy(w2_hbm, w2_v)
    lanes = lax.iota(jnp.int32, 16)
    valid = lanes < k

    def fire(c, s):
        idx_v, g2r_v, fr_v, semg, semf = sets[s]
        rowbase = base0 + c * ch
        pltpu.sync_copy(idx_hbm.at[pl.ds(rowbase * k, ch * k)], idx_v)
        pltpu.async_copy(g2_hbm.at[idx_v], g2r_v, semg)
        pltpu.async_copy(featsp_hbm.at[idx_v], fr_v, semf)

    def consume(c, s):
        idx_v, g2r_v, fr_v, semg, semf = sets[s]
        rowbase = base0 + c * ch
        pltpu.sync_copy(base2_hbm.at[pl.ds(rowbase, ch)], b_v)
        pltpu.sync_copy(feats_hbm.at[pl.ds(rowbase, ch)], fs_v)
        pltpu.make_async_copy(g2_hbm.at[idx_v], g2r_v, semg).wait()
        pltpu.make_async_copy(featsp_hbm.at[idx_v], fr_v, semf).wait()

        def row_work(i):
            accs = [jnp.zeros((16,), jnp.float32) for _ in range(k)]
            for hb in range(h_dim // 16):
                bvec = b_v[i, pl.ds(hb * 16, 16)]
                wvec = w2_v[pl.ds(hb * 16, 16)]
                for j in range(k):
                    gvec = g2r_v[i * k + j, pl.ds(hb * 16, 16)]
                    accs[j] = accs[j] + (jnp.maximum(bvec + gvec, 0.0)
                                         * wvec)
            sv = jnp.full((16,), -jnp.inf, jnp.float32)
            for j in range(k):
                sv = jnp.where(lanes == j, jnp.sum(accs[j]), sv)
            m = jnp.max(sv)
            e = jnp.where(valid, jnp.exp(sv - m), 0.0)
            wv = e / jnp.sum(e)
            plsc.store_compressed(wch_v.at[pl.ds(i * k, 16)], wv, mask=valid)
            wbs = [jnp.sum(jnp.where(lanes == j, wv, 0.0)) for j in range(k)]
            for db in range(d_dim // 16):
                accd = fs_v[i, pl.ds(db * 16, 16)]
                for j in range(k):
                    accd = accd + wbs[j] * fr_v[i * k + j, pl.ds(db * 16, 16)]
                out_v[i, pl.ds(db * 16, 16)] = accd

        def row_body(i2, _):
            row_work(i2 * 2)
            row_work(i2 * 2 + 1)
            return 0

        lax.fori_loop(0, ch // 2, row_body, 0)
        pltpu.sync_copy(out_v, enh_hbm.at[pl.ds(rowbase, ch)])
        pltpu.sync_copy(wch_v.at[pl.ds(0, ch * k)],
                        wout_hbm.at[pl.ds(rowbase * k, ch * k)])

    fire(0, 0)

    def pair_body(p, _):
        c0 = 2 * p
        fire(c0 + 1, 1)
        consume(c0, 0)

        @pl.when(c0 + 2 < nch)
        def _():
            fire(c0 + 2, 0)

        consume(c0 + 1, 1)
        return 0

    lax.fori_loop(0, nch // 2, pair_body, 0)


def kernel(object_features, language_embedding, centers, sizes, W1, b1, W2, b2):
    B, N, D = object_features.shape
    L = language_embedding.shape[1]
    H = b1.shape[0]
    K = min(5, N - 1)
    T = 256

    W1a = W1[:D]
    W1b = W1[D:2 * D]
    W1gp = W1[2 * D:2 * D + 3] / 5.0
    W1gs = W1[2 * D + 3:2 * D + 6] / 2.0
    W1l = W1[2 * D + 6:]
    b1r = b1.reshape(1, H)
    w2f = W2.reshape(H)
    lang3 = language_embedding.reshape(B, 1, L)

    NW = 32
    rows_per_w = (B * N) // NW
    CH = 16
    Dpad = D + (-D) % 128

    base2, g2, featsp, nidx, fidx = pl.pallas_call(
        _tc_kernel,
        grid=(B, N // T),
        in_specs=[
            pl.BlockSpec((1, T, D), lambda b, t: (b, t, 0)),
            pl.BlockSpec((1, 1, L), lambda b, t: (b, 0, 0)),
            pl.BlockSpec((1, T, 3), lambda b, t: (b, t, 0)),
            pl.BlockSpec((1, T, 3), lambda b, t: (b, t, 0)),
            pl.BlockSpec((1, N, 3), lambda b, t: (b, 0, 0)),
            pl.BlockSpec((D, H), lambda b, t: (0, 0)),
            pl.BlockSpec((D, H), lambda b, t: (0, 0)),
            pl.BlockSpec((3, H), lambda b, t: (0, 0)),
            pl.BlockSpec((3, H), lambda b, t: (0, 0)),
            pl.BlockSpec((L, H), lambda b, t: (0, 0)),
            pl.BlockSpec((1, H), lambda b, t: (0, 0)),
        ],
        out_specs=[
            pl.BlockSpec((1, T, H), lambda b, t: (b, t, 0)),
            pl.BlockSpec((1, T, H), lambda b, t: (b, t, 0)),
            pl.BlockSpec((1, T, Dpad), lambda b, t: (b, t, 0)),
            pl.BlockSpec((1, T, K), lambda b, t: (b, t, 0)),
            pl.BlockSpec((1, T, K), lambda b, t: (b, t, 0)),
        ],
        out_shape=[
            jax.ShapeDtypeStruct((B, N, H), jnp.float32),
            jax.ShapeDtypeStruct((B, N, H), jnp.float32),
            jax.ShapeDtypeStruct((B, N, Dpad), jnp.float32),
            jax.ShapeDtypeStruct((B, N, K), jnp.int32),
            jax.ShapeDtypeStruct((B, N, K), jnp.int32),
        ],
    )(object_features, lang3, centers, sizes, centers,
      W1a, W1b, W1gp, W1gs, W1l, b1r)

    feats_flat = object_features.reshape(B * N, D)
    feats_pad = featsp.reshape(B * N, Dpad)
    mesh = plsc.VectorSubcoreMesh(core_axis_name="c", subcore_axis_name="s")
    sc = functools.partial(
        pl.kernel,
        mesh=mesh,
        out_type=[
            jax.ShapeDtypeStruct((B * N, D), jnp.float32),
            jax.ShapeDtypeStruct((B * N * K,), jnp.float32),
        ],
        scratch_types=[
            pltpu.VMEM((CH * K,), jnp.int32),
            pltpu.VMEM((CH * K,), jnp.int32),
            pltpu.VMEM((CH * K, H), jnp.float32),
            pltpu.VMEM((CH * K, H), jnp.float32),
            pltpu.VMEM((CH * K, Dpad), jnp.float32),
            pltpu.VMEM((CH * K, Dpad), jnp.float32),
            pltpu.VMEM((CH, H), jnp.float32),
            pltpu.VMEM((CH, D), jnp.float32),
            pltpu.VMEM((CH, D), jnp.float32),
            pltpu.VMEM((CH * K + 16,), jnp.float32),
            pltpu.VMEM((H,), jnp.float32),
            pltpu.SemaphoreType.DMA,
            pltpu.SemaphoreType.DMA,
            pltpu.SemaphoreType.DMA,
            pltpu.SemaphoreType.DMA,
        ],
        compiler_params=pltpu.CompilerParams(needs_layout_passes=False),
    )(functools.partial(_sc_body, rows_per_w=rows_per_w, ch=CH,
                        h_dim=H, d_dim=D, k=K))

    enh_flat, w_flat = sc(
        g2.reshape(B * N, H),
        feats_flat,
        feats_pad,
        base2.reshape(B * N, H),
        fidx.reshape(B * N * K),
        w2f,
    )
    return enh_flat.reshape(B, N, D), w_flat.reshape(B, N, K), nidx


# speedup vs baseline: 1.0601x; 1.0601x over previous
import functools

import jax
import jax.numpy as jnp
from jax import lax
from jax.experimental import pallas as pl
from jax.experimental.pallas import tpu as pltpu
from jax.experimental.pallas import tpu_sc as plsc


def _tc_kernel(feats_ref, lang_ref, centers_ref, sizes_ref, centersT_ref,
               w1a_ref, w1b_ref, w1gp_ref, w1gs_ref, w1l_ref, b1_ref,
               base2_ref, g2_ref, featsp_ref, idx_ref, fidx_ref):
    T = centers_ref.shape[1]
    N = centersT_ref.shape[2]
    K = idx_ref.shape[2]
    Dpad = featsp_ref.shape[2]
    b = pl.program_id(0)
    r0 = pl.program_id(1) * T

    f = feats_ref[0]
    fa = jnp.dot(f, w1a_ref[...], preferred_element_type=jnp.float32)
    fb = jnp.dot(f, w1b_ref[...], preferred_element_type=jnp.float32)
    c = centers_ref[0]
    s = sizes_ref[0]
    cs = c[:, 0:1] * w1gp_ref[0:1, :] + s[:, 0:1] * w1gs_ref[0:1, :]
    for dd in range(1, 3):
        cs = cs + c[:, dd:dd + 1] * w1gp_ref[dd:dd + 1, :]
        cs = cs + s[:, dd:dd + 1] * w1gs_ref[dd:dd + 1, :]
    lb = jnp.dot(lang_ref[0], w1l_ref[...],
                 preferred_element_type=jnp.float32)
    base2_ref[0] = fa + cs + (lb + b1_ref[...])
    g2_ref[0] = fb - cs
    featsp_ref[0] = jnp.concatenate(
        [f, jnp.zeros((T, Dpad - f.shape[1]), jnp.float32)], axis=1)

    acc = None
    for dd in range(3):
        diff = c[:, dd:dd + 1] - centersT_ref[0, dd:dd + 1, :]
        sq = diff * diff
        acc = sq if acc is None else acc + sq
    col = jax.lax.broadcasted_iota(jnp.int32, (T, N), 1)
    row_g = r0 + jax.lax.broadcasted_iota(jnp.int32, (T, N), 0)
    d = jnp.where(col == row_g, jnp.inf, acc)

    idxs = []
    for _ in range(K):
        m = jnp.min(d, axis=1, keepdims=True)
        cand = jnp.where(d == m, col, N)
        aj = jnp.min(cand, axis=1, keepdims=True)
        idxs.append(aj)
        d = jnp.where(col == aj, jnp.inf, d)
    idx = jnp.concatenate(idxs, axis=1)
    idx_ref[0] = idx
    fidx_ref[0] = idx + b * N


def _sc_body(g2_hbm, feats_hbm, featsp_hbm, base2_hbm, idx_hbm, w2_hbm,
             enh_hbm, wout_hbm,
             idx0_v, idx1_v, g2r0_v, g2r1_v, fr0_v, fr1_v,
             b_v, fs_v, out_v, wch_v, w2_v,
             semg0, semg1, semf0, semf1,
             *, rows_per_w, ch, h_dim, d_dim, k):
    nc = 2
    wid = lax.axis_index("s") * nc + lax.axis_index("c")
    base0 = wid * rows_per_w
    nch = rows_per_w // ch
    sets = ((idx0_v, g2r0_v, fr0_v, semg0, semf0),
            (idx1_v, g2r1_v, fr1_v, semg1, semf1))

    pltpu.sync_copy(w2_hbm, w2_v)
    lanes = lax.iota(jnp.int32, 16)
    valid = lanes < k

    def fire(c, s):
        idx_v, g2r_v, fr_v, semg, semf = sets[s]
        rowbase = base0 + c * ch
        pltpu.sync_copy(idx_hbm.at[pl.ds(rowbase * k, ch * k)], idx_v)
        pltpu.async_copy(g2_hbm.at[idx_v], g2r_v, semg)
        pltpu.async_copy(featsp_hbm.at[idx_v], fr_v, semf)

    def consume(c, s):
        idx_v, g2r_v, fr_v, semg, semf = sets[s]
        rowbase = base0 + c * ch
        pltpu.sync_copy(base2_hbm.at[pl.ds(rowbase, ch)], b_v)
        pltpu.sync_copy(feats_hbm.at[pl.ds(rowbase, ch)], fs_v)
        pltpu.make_async_copy(g2_hbm.at[idx_v], g2r_v, semg).wait()
        pltpu.make_async_copy(featsp_hbm.at[idx_v], fr_v, semf).wait()

        def row_work(i):
            accs = [jnp.zeros((16,), jnp.float32) for _ in range(k)]
            for hb in range(h_dim // 16):
                bvec = b_v[i, pl.ds(hb * 16, 16)]
                wvec = w2_v[pl.ds(hb * 16, 16)]
                for j in range(k):
                    gvec = g2r_v[i * k + j, pl.ds(hb * 16, 16)]
                    accs[j] = accs[j] + (jnp.maximum(bvec + gvec, 0.0)
                                         * wvec)
            sv = jnp.full((16,), -jnp.inf, jnp.float32)
            for j in range(k):
                sv = jnp.where(lanes == j, jnp.sum(accs[j]), sv)
            m = jnp.max(sv)
            e = jnp.where(valid, jnp.exp(sv - m), 0.0)
            wv = e / jnp.sum(e)
            plsc.store_compressed(wch_v.at[pl.ds(i * k, 16)], wv, mask=valid)
            wbs = [jnp.sum(jnp.where(lanes == j, wv, 0.0)) for j in range(k)]
            for db in range(d_dim // 16):
                accd = fs_v[i, pl.ds(db * 16, 16)]
                for j in range(k):
                    accd = accd + wbs[j] * fr_v[i * k + j, pl.ds(db * 16, 16)]
                out_v[i, pl.ds(db * 16, 16)] = accd

        def row_body(i, _):
            row_work(i)
            return 0

        lax.fori_loop(0, ch, row_body, 0)
        pltpu.sync_copy(out_v, enh_hbm.at[pl.ds(rowbase, ch)])
        pltpu.sync_copy(wch_v.at[pl.ds(0, ch * k)],
                        wout_hbm.at[pl.ds(rowbase * k, ch * k)])

    fire(0, 0)

    def pair_body(p, _):
        c0 = 2 * p
        fire(c0 + 1, 1)
        consume(c0, 0)

        @pl.when(c0 + 2 < nch)
        def _():
            fire(c0 + 2, 0)

        consume(c0 + 1, 1)
        return 0

    lax.fori_loop(0, nch // 2, pair_body, 0)


def kernel(object_features, language_embedding, centers, sizes, W1, b1, W2, b2):
    B, N, D = object_features.shape
    L = language_embedding.shape[1]
    H = b1.shape[0]
    K = min(5, N - 1)
    T = 256

    W1a = W1[:D]
    W1b = W1[D:2 * D]
    W1gp = W1[2 * D:2 * D + 3] / 5.0
    W1gs = W1[2 * D + 3:2 * D + 6] / 2.0
    W1l = W1[2 * D + 6:]
    b1r = b1.reshape(1, H)
    w2f = W2.reshape(H)
    lang3 = language_embedding.reshape(B, 1, L)
    centersT = jnp.swapaxes(centers, 1, 2)

    NW = 32
    rows_per_w = (B * N) // NW
    CH = 16
    Dpad = D + (-D) % 128

    base2, g2, featsp, nidx, fidx = pl.pallas_call(
        _tc_kernel,
        grid=(B, N // T),
        in_specs=[
            pl.BlockSpec((1, T, D), lambda b, t: (b, t, 0)),
            pl.BlockSpec((1, 1, L), lambda b, t: (b, 0, 0)),
            pl.BlockSpec((1, T, 3), lambda b, t: (b, t, 0)),
            pl.BlockSpec((1, T, 3), lambda b, t: (b, t, 0)),
            pl.BlockSpec((1, 3, N), lambda b, t: (b, 0, 0)),
            pl.BlockSpec((D, H), lambda b, t: (0, 0)),
            pl.BlockSpec((D, H), lambda b, t: (0, 0)),
            pl.BlockSpec((3, H), lambda b, t: (0, 0)),
            pl.BlockSpec((3, H), lambda b, t: (0, 0)),
            pl.BlockSpec((L, H), lambda b, t: (0, 0)),
            pl.BlockSpec((1, H), lambda b, t: (0, 0)),
        ],
        out_specs=[
            pl.BlockSpec((1, T, H), lambda b, t: (b, t, 0)),
            pl.BlockSpec((1, T, H), lambda b, t: (b, t, 0)),
            pl.BlockSpec((1, T, Dpad), lambda b, t: (b, t, 0)),
            pl.BlockSpec((1, T, K), lambda b, t: (b, t, 0)),
            pl.BlockSpec((1, T, K), lambda b, t: (b, t, 0)),
        ],
        out_shape=[
            jax.ShapeDtypeStruct((B, N, H), jnp.float32),
            jax.ShapeDtypeStruct((B, N, H), jnp.float32),
            jax.ShapeDtypeStruct((B, N, Dpad), jnp.float32),
            jax.ShapeDtypeStruct((B, N, K), jnp.int32),
            jax.ShapeDtypeStruct((B, N, K), jnp.int32),
        ],
    )(object_features, lang3, centers, sizes, centersT,
      W1a, W1b, W1gp, W1gs, W1l, b1r)

    feats_flat = object_features.reshape(B * N, D)
    feats_pad = featsp.reshape(B * N, Dpad)
    mesh = plsc.VectorSubcoreMesh(core_axis_name="c", subcore_axis_name="s")
    sc = functools.partial(
        pl.kernel,
        mesh=mesh,
        out_type=[
            jax.ShapeDtypeStruct((B * N, D), jnp.float32),
            jax.ShapeDtypeStruct((B * N * K,), jnp.float32),
        ],
        scratch_types=[
            pltpu.VMEM((CH * K,), jnp.int32),
            pltpu.VMEM((CH * K,), jnp.int32),
            pltpu.VMEM((CH * K, H), jnp.float32),
            pltpu.VMEM((CH * K, H), jnp.float32),
            pltpu.VMEM((CH * K, Dpad), jnp.float32),
            pltpu.VMEM((CH * K, Dpad), jnp.float32),
            pltpu.VMEM((CH, H), jnp.float32),
            pltpu.VMEM((CH, D), jnp.float32),
            pltpu.VMEM((CH, D), jnp.float32),
            pltpu.VMEM((CH * K + 16,), jnp.float32),
            pltpu.VMEM((H,), jnp.float32),
            pltpu.SemaphoreType.DMA,
            pltpu.SemaphoreType.DMA,
            pltpu.SemaphoreType.DMA,
            pltpu.SemaphoreType.DMA,
        ],
        compiler_params=pltpu.CompilerParams(needs_layout_passes=False),
    )(functools.partial(_sc_body, rows_per_w=rows_per_w, ch=CH,
                        h_dim=H, d_dim=D, k=K))

    enh_flat, w_flat = sc(
        g2.reshape(B * N, H),
        feats_flat,
        feats_pad,
        base2.reshape(B * N, H),
        fidx.reshape(B * N * K),
        w2f,
    )
    return enh_flat.reshape(B, N, D), w_flat.reshape(B, N, K), nidx


# speedup vs baseline: 1.0782x; 1.0171x over previous
import functools

import jax
import jax.numpy as jnp
from jax import lax
from jax.experimental import pallas as pl
from jax.experimental.pallas import tpu as pltpu
from jax.experimental.pallas import tpu_sc as plsc


def _tc_kernel(feats_ref, lang_ref, centers_ref, sizes_ref, centersT_ref,
               w1a_ref, w1b_ref, w1gp_ref, w1gs_ref, w1l_ref, b1_ref,
               base2_ref, g2_ref, featsp_ref, idx_ref, fidx_ref):
    T = centers_ref.shape[1]
    N = centersT_ref.shape[2]
    K = idx_ref.shape[2]
    Dpad = featsp_ref.shape[2]
    b = pl.program_id(0)
    r0 = pl.program_id(1) * T

    f = feats_ref[0]
    fa = jnp.dot(f, w1a_ref[...], preferred_element_type=jnp.float32)
    fb = jnp.dot(f, w1b_ref[...], preferred_element_type=jnp.float32)
    c = centers_ref[0]
    s = sizes_ref[0]
    cs = c[:, 0:1] * w1gp_ref[0:1, :] + s[:, 0:1] * w1gs_ref[0:1, :]
    for dd in range(1, 3):
        cs = cs + c[:, dd:dd + 1] * w1gp_ref[dd:dd + 1, :]
        cs = cs + s[:, dd:dd + 1] * w1gs_ref[dd:dd + 1, :]
    lb = jnp.dot(lang_ref[0], w1l_ref[...],
                 preferred_element_type=jnp.float32)
    base2_ref[0] = fa + cs + (lb + b1_ref[...])
    g2_ref[0] = fb - cs
    featsp_ref[0] = jnp.concatenate(
        [f, jnp.zeros((T, Dpad - f.shape[1]), jnp.float32)], axis=1)

    acc = None
    for dd in range(3):
        diff = c[:, dd:dd + 1] - centersT_ref[0, dd:dd + 1, :]
        sq = diff * diff
        acc = sq if acc is None else acc + sq
    col = jax.lax.broadcasted_iota(jnp.int32, (T, N), 1)
    row_g = r0 + jax.lax.broadcasted_iota(jnp.int32, (T, N), 0)
    d = jnp.where(col == row_g, jnp.inf, acc)

    idxs = []
    for _ in range(K):
        m = jnp.min(d, axis=1, keepdims=True)
        cand = jnp.where(d == m, col, N)
        aj = jnp.min(cand, axis=1, keepdims=True)
        idxs.append(aj)
        d = jnp.where(col == aj, jnp.inf, d)
    idx = jnp.concatenate(idxs, axis=1)
    idx_ref[0] = idx
    fidx_ref[0] = idx + b * N


def _sc_body(g2_hbm, feats_hbm, featsp_hbm, base2_hbm, idx_hbm, w2_hbm,
             enh_hbm, wout_hbm,
             idx0_v, idx1_v, g2r0_v, g2r1_v, fr0_v, fr1_v,
             b_v, fs_v, out_v, wch_v, w2_v,
             semg0, semg1, semf0, semf1,
             *, rows_per_w, ch, h_dim, d_dim, k):
    nc = 2
    wid = lax.axis_index("s") * nc + lax.axis_index("c")
    base0 = wid * rows_per_w
    nch = rows_per_w // ch
    sets = ((idx0_v, g2r0_v, fr0_v, semg0, semf0),
            (idx1_v, g2r1_v, fr1_v, semg1, semf1))

    pltpu.sync_copy(w2_hbm, w2_v)
    lanes = lax.iota(jnp.int32, 16)
    valid = lanes < k

    def fire(c, s):
        idx_v, g2r_v, fr_v, semg, semf = sets[s]
        rowbase = base0 + c * ch
        pltpu.sync_copy(idx_hbm.at[pl.ds(rowbase * k, ch * k)], idx_v)
        pltpu.async_copy(g2_hbm.at[idx_v], g2r_v, semg)
        pltpu.async_copy(featsp_hbm.at[idx_v], fr_v, semf)

    def consume(c, s):
        idx_v, g2r_v, fr_v, semg, semf = sets[s]
        rowbase = base0 + c * ch
        pltpu.sync_copy(base2_hbm.at[pl.ds(rowbase, ch)], b_v)
        pltpu.sync_copy(feats_hbm.at[pl.ds(rowbase, ch)], fs_v)
        pltpu.make_async_copy(g2_hbm.at[idx_v], g2r_v, semg).wait()
        pltpu.make_async_copy(featsp_hbm.at[idx_v], fr_v, semf).wait()

        def row_work(i):
            accs = [[jnp.zeros((16,), jnp.float32),
                     jnp.zeros((16,), jnp.float32)] for _ in range(k)]
            for hb in range(h_dim // 16):
                bvec = b_v[i, pl.ds(hb * 16, 16)]
                wvec = w2_v[pl.ds(hb * 16, 16)]
                p = hb & 1
                for j in range(k):
                    gvec = g2r_v[i * k + j, pl.ds(hb * 16, 16)]
                    accs[j][p] = accs[j][p] + (jnp.maximum(bvec + gvec, 0.0)
                                               * wvec)
            sv = jnp.full((16,), -jnp.inf, jnp.float32)
            for j in range(k):
                sv = jnp.where(lanes == j, jnp.sum(accs[j][0] + accs[j][1]),
                               sv)
            m = jnp.max(sv)
            e = jnp.where(valid, jnp.exp(sv - m), 0.0)
            wv = e / jnp.sum(e)
            plsc.store_compressed(wch_v.at[pl.ds(i * k, 16)], wv, mask=valid)
            wbs = [jnp.sum(jnp.where(lanes == j, wv, 0.0)) for j in range(k)]
            for db in range(d_dim // 16):
                terms = [fs_v[i, pl.ds(db * 16, 16)]]
                terms += [wbs[j] * fr_v[i * k + j, pl.ds(db * 16, 16)]
                          for j in range(k)]
                while len(terms) > 1:
                    terms = ([terms[t] + terms[t + 1]
                              for t in range(0, len(terms) - 1, 2)]
                             + ([terms[-1]] if len(terms) % 2 else []))
                out_v[i, pl.ds(db * 16, 16)] = terms[0]

        def row_body(i, _):
            row_work(i)
            return 0

        lax.fori_loop(0, ch, row_body, 0)
        pltpu.sync_copy(out_v, enh_hbm.at[pl.ds(rowbase, ch)])
        pltpu.sync_copy(wch_v.at[pl.ds(0, ch * k)],
                        wout_hbm.at[pl.ds(rowbase * k, ch * k)])

    fire(0, 0)

    def pair_body(p, _):
        c0 = 2 * p
        fire(c0 + 1, 1)
        consume(c0, 0)

        @pl.when(c0 + 2 < nch)
        def _():
            fire(c0 + 2, 0)

        consume(c0 + 1, 1)
        return 0

    lax.fori_loop(0, nch // 2, pair_body, 0)


def kernel(object_features, language_embedding, centers, sizes, W1, b1, W2, b2):
    B, N, D = object_features.shape
    L = language_embedding.shape[1]
    H = b1.shape[0]
    K = min(5, N - 1)
    T = 256

    W1a = W1[:D]
    W1b = W1[D:2 * D]
    W1gp = W1[2 * D:2 * D + 3] / 5.0
    W1gs = W1[2 * D + 3:2 * D + 6] / 2.0
    W1l = W1[2 * D + 6:]
    b1r = b1.reshape(1, H)
    w2f = W2.reshape(H)
    lang3 = language_embedding.reshape(B, 1, L)
    centersT = jnp.swapaxes(centers, 1, 2)

    NW = 32
    rows_per_w = (B * N) // NW
    CH = 16
    Dpad = D + (-D) % 128

    base2, g2, featsp, nidx, fidx = pl.pallas_call(
        _tc_kernel,
        grid=(B, N // T),
        in_specs=[
            pl.BlockSpec((1, T, D), lambda b, t: (b, t, 0)),
            pl.BlockSpec((1, 1, L), lambda b, t: (b, 0, 0)),
            pl.BlockSpec((1, T, 3), lambda b, t: (b, t, 0)),
            pl.BlockSpec((1, T, 3), lambda b, t: (b, t, 0)),
            pl.BlockSpec((1, 3, N), lambda b, t: (b, 0, 0)),
            pl.BlockSpec((D, H), lambda b, t: (0, 0)),
            pl.BlockSpec((D, H), lambda b, t: (0, 0)),
            pl.BlockSpec((3, H), lambda b, t: (0, 0)),
            pl.BlockSpec((3, H), lambda b, t: (0, 0)),
            pl.BlockSpec((L, H), lambda b, t: (0, 0)),
            pl.BlockSpec((1, H), lambda b, t: (0, 0)),
        ],
        out_specs=[
            pl.BlockSpec((1, T, H), lambda b, t: (b, t, 0)),
            pl.BlockSpec((1, T, H), lambda b, t: (b, t, 0)),
            pl.BlockSpec((1, T, Dpad), lambda b, t: (b, t, 0)),
            pl.BlockSpec((1, T, K), lambda b, t: (b, t, 0)),
            pl.BlockSpec((1, T, K), lambda b, t: (b, t, 0)),
        ],
        out_shape=[
            jax.ShapeDtypeStruct((B, N, H), jnp.float32),
            jax.ShapeDtypeStruct((B, N, H), jnp.float32),
            jax.ShapeDtypeStruct((B, N, Dpad), jnp.float32),
            jax.ShapeDtypeStruct((B, N, K), jnp.int32),
            jax.ShapeDtypeStruct((B, N, K), jnp.int32),
        ],
    )(object_features, lang3, centers, sizes, centersT,
      W1a, W1b, W1gp, W1gs, W1l, b1r)

    feats_flat = object_features.reshape(B * N, D)
    feats_pad = featsp.reshape(B * N, Dpad)
    mesh = plsc.VectorSubcoreMesh(core_axis_name="c", subcore_axis_name="s")
    sc = functools.partial(
        pl.kernel,
        mesh=mesh,
        out_type=[
            jax.ShapeDtypeStruct((B * N, D), jnp.float32),
            jax.ShapeDtypeStruct((B * N * K,), jnp.float32),
        ],
        scratch_types=[
            pltpu.VMEM((CH * K,), jnp.int32),
            pltpu.VMEM((CH * K,), jnp.int32),
            pltpu.VMEM((CH * K, H), jnp.float32),
            pltpu.VMEM((CH * K, H), jnp.float32),
            pltpu.VMEM((CH * K, Dpad), jnp.float32),
            pltpu.VMEM((CH * K, Dpad), jnp.float32),
            pltpu.VMEM((CH, H), jnp.float32),
            pltpu.VMEM((CH, D), jnp.float32),
            pltpu.VMEM((CH, D), jnp.float32),
            pltpu.VMEM((CH * K + 16,), jnp.float32),
            pltpu.VMEM((H,), jnp.float32),
            pltpu.SemaphoreType.DMA,
            pltpu.SemaphoreType.DMA,
            pltpu.SemaphoreType.DMA,
            pltpu.SemaphoreType.DMA,
        ],
        compiler_params=pltpu.CompilerParams(needs_layout_passes=False),
    )(functools.partial(_sc_body, rows_per_w=rows_per_w, ch=CH,
                        h_dim=H, d_dim=D, k=K))

    enh_flat, w_flat = sc(
        g2.reshape(B * N, H),
        feats_flat,
        feats_pad,
        base2.reshape(B * N, H),
        fidx.reshape(B * N * K),
        w2f,
    )
    return enh_flat.reshape(B, N, D), w_flat.reshape(B, N, K), nidx
